# Initial kernel scaffold; baseline (speedup 1.0000x reference)
import jax
import jax.numpy as jnp
from jax.experimental import pallas as pl


def kernel(x, c1w, c1b, c2w, c2b, f1w, f1b, f2w, f2b, f3w, f3b):
    raise NotImplementedError("write your pallas kernel here")


# speedup vs baseline: 10.6157x; 10.6157x over previous
import jax
import jax.numpy as jnp
from jax import lax
from jax.experimental import pallas as pl
from jax.experimental.pallas import tpu as pltpu

CI1, CO1, H, W, K5 = 3, 12, 32, 32, 5
OH1 = H - K5 + 1
PH1 = OH1 // 2
CI2, CO2 = CO1, 64
OH2 = PH1 - K5 + 1
PH2 = OH2 // 2
FLAT = CO2 * PH2 * PH2
N1, N2, N3 = 512, 128, 10

BT = 128

M1 = CO1 * OH1
K1 = K5 * W
M2 = CO2 * OH2
K2 = K5 * CI2 * PH1

_HI = lax.Precision.HIGHEST
_DN = (((1,), (0,)), ((), ()))
_DT = (((0,), (0,)), ((), ()))


def _mm(a, b):
    return lax.dot_general(a, b, _DN, precision=_HI,
                           preferred_element_type=jnp.float32)


def _lenet_kernel(x_ref, t1_ref, b1_ref, t2_ref, b2_ref,
                  w1_ref, fb1_ref, w2_ref, fb2_ref, w3_ref, fb3_ref,
                  out_ref, xt_ref, p1_ref, p2_ref):
    f32 = jnp.float32

    ii = lax.broadcasted_iota(jnp.int32, (BT, BT), 0)
    jj = lax.broadcasted_iota(jnp.int32, (BT, BT), 1)
    eye = (ii == jj).astype(f32)
    for ci in range(CI1):
        xs = x_ref[:, H * W * ci:H * W * (ci + 1)]
        xt_ref[H * W * ci:H * W * (ci + 1), :] = lax.dot_general(
            xs, eye, _DT, precision=_HI, preferred_element_type=f32)

    b1 = b1_ref[...]

    def conv1_row(oh):
        a0 = W * oh
        acc = _mm(t1_ref[0], xt_ref[a0:a0 + K1, :])
        acc += _mm(t1_ref[1], xt_ref[H * W + a0:H * W + a0 + K1, :])
        acc += _mm(t1_ref[2], xt_ref[2 * H * W + a0:2 * H * W + a0 + K1, :])
        return acc + b1

    R1 = CO1 * PH1
    for ph in range(PH1):
        row = jnp.maximum(conv1_row(2 * ph), conv1_row(2 * ph + 1))
        row = jnp.max(row.reshape(R1, 2, BT), axis=1)
        p1_ref[R1 * ph:R1 * (ph + 1), :] = row

    b2 = b2_ref[...]

    def conv2_row(oh):
        return _mm(t2_ref[...], p1_ref[R1 * oh:R1 * oh + K2, :]) + b2

    R2 = CO2 * PH2
    for ph in range(PH2):
        row = jnp.maximum(conv2_row(2 * ph), conv2_row(2 * ph + 1))
        row = jnp.max(row.reshape(R2, 2, BT), axis=1)
        p2_ref[R2 * ph:R2 * (ph + 1), :] = row

    flat = p2_ref[...]

    h = jnp.maximum(_mm(w1_ref[...], flat) + fb1_ref[...], 0.0)
    h = jnp.maximum(_mm(w2_ref[...], h) + fb2_ref[...], 0.0)
    h = jnp.maximum(_mm(w3_ref[...], h) + fb3_ref[...], 0.0)
    out_ref[...] = h


def kernel(x, c1w, c1b, c2w, c2b, f1w, f1b, f2w, f2b, f3w, f3b):
    B = x.shape[0]
    f32 = jnp.float32

    def band(ow, kw, iw):
        o = lax.broadcasted_iota(jnp.int32, (ow, kw, iw), 0)
        k = lax.broadcasted_iota(jnp.int32, (ow, kw, iw), 1)
        i = lax.broadcasted_iota(jnp.int32, (ow, kw, iw), 2)
        return (i == o + k).astype(f32)

    t1 = jnp.einsum('abde,wei->bawdi', c1w, band(OH1, K5, W))
    t1 = t1.reshape(CI1, M1, K1)
    b1r = jnp.repeat(c1b, OH1).reshape(M1, 1)

    t2 = jnp.einsum('abde,wei->awdbi', c2w, band(OH2, K5, PH1))
    t2 = t2.reshape(M2, K2)
    b2r = jnp.repeat(c2b, OH2).reshape(M2, 1)

    w1r = f1w.reshape(N1, CO2, PH2, PH2).transpose(0, 2, 1, 3).reshape(N1, FLAT)

    xf = x.reshape(B, CI1 * H * W)
    nt = B // BT

    out = pl.pallas_call(
        _lenet_kernel,
        out_shape=jax.ShapeDtypeStruct((N3, B), f32),
        grid_spec=pltpu.PrefetchScalarGridSpec(
            num_scalar_prefetch=0,
            grid=(nt,),
            in_specs=[
                pl.BlockSpec((BT, CI1 * H * W), lambda i: (i, 0)),
                pl.BlockSpec((CI1, M1, K1), lambda i: (0, 0, 0)),
                pl.BlockSpec((M1, 1), lambda i: (0, 0)),
                pl.BlockSpec((M2, K2), lambda i: (0, 0)),
                pl.BlockSpec((M2, 1), lambda i: (0, 0)),
                pl.BlockSpec((N1, FLAT), lambda i: (0, 0)),
                pl.BlockSpec((N1, 1), lambda i: (0, 0)),
                pl.BlockSpec((N2, N1), lambda i: (0, 0)),
                pl.BlockSpec((N2, 1), lambda i: (0, 0)),
                pl.BlockSpec((N3, N2), lambda i: (0, 0)),
                pl.BlockSpec((N3, 1), lambda i: (0, 0)),
            ],
            out_specs=pl.BlockSpec((N3, BT), lambda i: (0, i)),
            scratch_shapes=[
                pltpu.VMEM((CI1 * H * W, BT), f32),
                pltpu.VMEM((PH1 * CI2 * PH1, BT), f32),
                pltpu.VMEM((FLAT, BT), f32),
            ],
        ),
        compiler_params=pltpu.CompilerParams(
            dimension_semantics=("parallel",),
            vmem_limit_bytes=64 * 1024 * 1024),
    )(xf, t1, b1r, t2, b2r,
      w1r, f1b.reshape(N1, 1), f2w, f2b.reshape(N2, 1),
      f3w, f3b.reshape(N3, 1))
    return out.T
